# Initial kernel scaffold; baseline (speedup 1.0000x reference)
import jax
import jax.numpy as jnp
from jax.experimental import pallas as pl


def kernel(x, edge_index, W1, b1, W2, b2, Wfc, bfc):
    raise NotImplementedError("write your pallas kernel here")


# speedup vs baseline: 179.7724x; 179.7724x over previous
import functools

import jax
import jax.numpy as jnp
from jax import lax
from jax.experimental import pallas as pl
from jax.experimental.pallas import tpu as pltpu
from jax.experimental.pallas import tpu_sc as plsc

N = 500
NPAD = 512
F = 50
C_IN = 128
E = 8000
NW = 32
EPW = 256
EPAD = NW * EPW
ACELLS = NPAD * NPAD
SLICE = ACELLS // 16


def _sc_build_counts(dst_hbm, src_hbm, out_hbm, shared, dvec, svec,
                     idx_a, idx_b, val_a, val_b, zbuf):
    c = lax.axis_index("c")
    s = lax.axis_index("s")
    wid = c * 16 + s

    def zbody(i, carry):
        zbuf[pl.ds(i * 16, 16)] = jnp.zeros((16,), jnp.float32)
        return carry
    lax.fori_loop(0, SLICE // 16, zbody, 0)
    pltpu.sync_copy(zbuf, shared.at[pl.ds(s * SLICE, SLICE)])
    plsc.subcore_barrier()

    base = wid * EPW
    pltpu.sync_copy(dst_hbm.at[pl.ds(base, EPW)], dvec)
    pltpu.sync_copy(src_hbm.at[pl.ds(base, EPW)], svec)

    lane = lax.iota(jnp.int32, 16)
    for j in range(EPW // 16):
        d = dvec[pl.ds(j * 16, 16)]
        sv = svec[pl.ds(j * 16, 16)]
        flat = d * NPAD + sv
        gid = base + j * 16 + lane
        val = jnp.where(gid < E, jnp.float32(1.0), jnp.float32(0.0))
        if j < 8:
            idx_a[pl.ds(j * 16, 16)] = flat
            val_a[pl.ds(j * 16, 16)] = val
        else:
            idx_b[pl.ds((j - 8) * 16, 16)] = flat
            val_b[pl.ds((j - 8) * 16, 16)] = val

    pltpu.sync_copy(val_a, shared.at[idx_a], add=True)
    pltpu.sync_copy(val_b, shared.at[idx_b], add=True)
    plsc.subcore_barrier()

    pltpu.sync_copy(shared.at[pl.ds(s * SLICE, SLICE)],
                    out_hbm.at[c, pl.ds(s * SLICE, SLICE)])


@jax.jit
def _build_counts(dst, src):
    mesh = plsc.VectorSubcoreMesh(core_axis_name="c", subcore_axis_name="s")
    return pl.kernel(
        _sc_build_counts,
        out_type=jax.ShapeDtypeStruct((2, ACELLS), jnp.float32),
        mesh=mesh,
        scratch_types=[
            pltpu.MemorySpace.VMEM_SHARED((ACELLS,), jnp.float32),
            pltpu.VMEM((EPW,), jnp.int32),
            pltpu.VMEM((EPW,), jnp.int32),
            pltpu.VMEM((128,), jnp.int32),
            pltpu.VMEM((128,), jnp.int32),
            pltpu.VMEM((128,), jnp.float32),
            pltpu.VMEM((128,), jnp.float32),
            pltpu.VMEM((SLICE,), jnp.float32),
        ],
    )(dst, src)


def _tc_main(a_ref, x_ref, w1_ref, b1_ref, w2_ref, b2_ref, wfc_ref, bfc_ref,
             out_ref, ahat_s, acc_s):
    f = pl.program_id(0)

    @pl.when(f == 0)
    def _init():
        A = a_ref[0] + a_ref[1]
        deg = jnp.sum(A, axis=1) + 1.0
        dinv = lax.rsqrt(deg)
        r = lax.broadcasted_iota(jnp.int32, (NPAD, NPAD), 0)
        cc = lax.broadcasted_iota(jnp.int32, (NPAD, NPAD), 1)
        eye = jnp.where(r == cc, jnp.float32(1.0), jnp.float32(0.0))
        ahat_s[...] = (A + eye) * dinv[:, None] * dinv[None, :]
        acc_s[...] = jnp.zeros((1, NPAD), jnp.float32)

    ah = ahat_s[...]
    xp = x_ref[0]
    t0 = jnp.dot(ah, xp, preferred_element_type=jnp.float32)
    h1 = jnp.maximum(
        jnp.dot(t0, w1_ref[...], preferred_element_type=jnp.float32)
        + b1_ref[...], 0.0)
    t1 = jnp.dot(ah, h1, preferred_element_type=jnp.float32)
    h2 = jnp.maximum(
        jnp.dot(t1, w2_ref[...], preferred_element_type=jnp.float32)
        + b2_ref[...], 0.0)
    acc_s[...] += jnp.sum(h2[:N, :], axis=0)[None, :]

    @pl.when(f == F - 1)
    def _fin():
        hmean = acc_s[...] / jnp.float32(N * F)
        out_ref[...] = (
            jnp.dot(hmean, wfc_ref[...], preferred_element_type=jnp.float32)
            + bfc_ref[...])


@jax.jit
def _main(a2, xp, W1, b1, W2, b2, Wfc, bfc):
    const3 = lambda f: (0, 0, 0)
    const2 = lambda f: (0, 0)
    return pl.pallas_call(
        _tc_main,
        grid=(F,),
        in_specs=[
            pl.BlockSpec((2, NPAD, NPAD), const3),
            pl.BlockSpec((1, NPAD, C_IN), lambda f: (f, 0, 0)),
            pl.BlockSpec(W1.shape, const2),
            pl.BlockSpec(b1.shape, const2),
            pl.BlockSpec(W2.shape, const2),
            pl.BlockSpec(b2.shape, const2),
            pl.BlockSpec(Wfc.shape, const2),
            pl.BlockSpec(bfc.shape, const2),
        ],
        out_specs=pl.BlockSpec((1, NPAD), const2),
        out_shape=jax.ShapeDtypeStruct((1, NPAD), jnp.float32),
        scratch_shapes=[
            pltpu.VMEM((NPAD, NPAD), jnp.float32),
            pltpu.VMEM((1, NPAD), jnp.float32),
        ],
        compiler_params=pltpu.CompilerParams(
            dimension_semantics=("arbitrary",)),
    )(a2, xp, W1, b1, W2, b2, Wfc, bfc)


def kernel(x, edge_index, W1, b1, W2, b2, Wfc, bfc):
    ei = jnp.pad(edge_index, ((0, 0), (0, EPAD - E)))
    counts2 = _build_counts(ei[1], ei[0])
    a2 = counts2.reshape(2, NPAD, NPAD)
    xb = x.reshape(F, N, C_IN)
    xp = jnp.pad(xb, ((0, 0), (0, NPAD - N), (0, 0)))
    out = _main(a2, xp, W1, b1.reshape(1, -1), W2, b2.reshape(1, -1),
                Wfc, bfc.reshape(1, -1))
    return out.reshape(-1)[: Wfc.shape[1]]


# speedup vs baseline: 250.3131x; 1.3924x over previous
import functools

import jax
import jax.numpy as jnp
from jax import lax
from jax.experimental import pallas as pl
from jax.experimental.pallas import tpu as pltpu
from jax.experimental.pallas import tpu_sc as plsc

N = 500
NPAD = 512
F = 50
C_IN = 128
E = 8000
NW = 32
EPW = 256
EPAD = NW * EPW
ACELLS = NPAD * NPAD
SLICE = ACELLS // 16


def _sc_build_counts(dst_hbm, src_hbm, out_hbm, shared, dvec, svec,
                     idx_a, idx_b, val_a, val_b, zbuf):
    c = lax.axis_index("c")
    s = lax.axis_index("s")
    wid = c * 16 + s

    def zbody(i, carry):
        zbuf[pl.ds(i * 16, 16)] = jnp.zeros((16,), jnp.float32)
        return carry
    lax.fori_loop(0, SLICE // 16, zbody, 0)
    pltpu.sync_copy(zbuf, shared.at[pl.ds(s * SLICE, SLICE)])
    plsc.subcore_barrier()

    base = wid * EPW
    pltpu.sync_copy(dst_hbm.at[pl.ds(base, EPW)], dvec)
    pltpu.sync_copy(src_hbm.at[pl.ds(base, EPW)], svec)

    lane = lax.iota(jnp.int32, 16)
    for j in range(EPW // 16):
        d = dvec[pl.ds(j * 16, 16)]
        sv = svec[pl.ds(j * 16, 16)]
        flat = d * NPAD + sv
        gid = base + j * 16 + lane
        val = jnp.where(gid < E, jnp.float32(1.0), jnp.float32(0.0))
        if j < 8:
            idx_a[pl.ds(j * 16, 16)] = flat
            val_a[pl.ds(j * 16, 16)] = val
        else:
            idx_b[pl.ds((j - 8) * 16, 16)] = flat
            val_b[pl.ds((j - 8) * 16, 16)] = val

    pltpu.sync_copy(val_a, shared.at[idx_a], add=True)
    pltpu.sync_copy(val_b, shared.at[idx_b], add=True)
    plsc.subcore_barrier()

    pltpu.sync_copy(shared.at[pl.ds(s * SLICE, SLICE)],
                    out_hbm.at[c, pl.ds(s * SLICE, SLICE)])


@jax.jit
def _build_counts(dst, src):
    mesh = plsc.VectorSubcoreMesh(core_axis_name="c", subcore_axis_name="s")
    return pl.kernel(
        _sc_build_counts,
        out_type=jax.ShapeDtypeStruct((2, ACELLS), jnp.float32),
        mesh=mesh,
        scratch_types=[
            pltpu.MemorySpace.VMEM_SHARED((ACELLS,), jnp.float32),
            pltpu.VMEM((EPW,), jnp.int32),
            pltpu.VMEM((EPW,), jnp.int32),
            pltpu.VMEM((128,), jnp.int32),
            pltpu.VMEM((128,), jnp.int32),
            pltpu.VMEM((128,), jnp.float32),
            pltpu.VMEM((128,), jnp.float32),
            pltpu.VMEM((SLICE,), jnp.float32),
        ],
    )(dst, src)


FB = 2


def _tc_main(a_ref, x_ref, w1_ref, b1_ref, w2_ref, b2_ref, wfc_ref, bfc_ref,
             out_ref, ahat_s, acc_s):
    f = pl.program_id(0)

    @pl.when(f == 0)
    def _init():
        A = a_ref[0] + a_ref[1]
        deg = jnp.sum(A, axis=1) + 1.0
        dinv = lax.rsqrt(deg)
        r = lax.broadcasted_iota(jnp.int32, (NPAD, NPAD), 0)
        cc = lax.broadcasted_iota(jnp.int32, (NPAD, NPAD), 1)
        eye = jnp.where(r == cc, jnp.float32(1.0), jnp.float32(0.0))
        ahat_s[...] = ((A + eye) * dinv[:, None] * dinv[None, :]).astype(
            jnp.bfloat16)
        acc_s[...] = jnp.zeros((1, NPAD), jnp.float32)

    ah = ahat_s[...]
    xcat = jnp.concatenate([x_ref[i] for i in range(FB)], axis=1)
    xcat = jnp.concatenate(
        [xcat, jnp.zeros((NPAD - N, FB * C_IN), jnp.float32)], axis=0)
    t0 = jnp.dot(ah, xcat.astype(jnp.bfloat16),
                 preferred_element_type=jnp.float32)
    w1 = w1_ref[...]
    b1 = b1_ref[...]
    h1 = [
        jnp.maximum(
            jnp.dot(t0[:, i * C_IN:(i + 1) * C_IN].astype(jnp.bfloat16), w1,
                    preferred_element_type=jnp.float32) + b1,
            0.0).astype(jnp.bfloat16)
        for i in range(FB)
    ]
    t1 = jnp.dot(ah, jnp.concatenate(h1, axis=1),
                 preferred_element_type=jnp.float32)
    w2 = w2_ref[...]
    b2 = b2_ref[...]
    part = jnp.zeros((1, NPAD), jnp.float32)
    for i in range(FB):
        h2 = jnp.maximum(
            jnp.dot(t1[:, i * 256:(i + 1) * 256].astype(jnp.bfloat16), w2,
                    preferred_element_type=jnp.float32) + b2, 0.0)
        part += jnp.sum(h2[:N, :], axis=0)[None, :]
    acc_s[...] += part

    @pl.when(f == F // FB - 1)
    def _fin():
        hmean = acc_s[...] / jnp.float32(N * F)
        out_ref[...] = (
            jnp.dot(hmean, wfc_ref[...], preferred_element_type=jnp.float32)
            + bfc_ref[...])


@jax.jit
def _main(a2, xb, W1, b1, W2, b2, Wfc, bfc):
    const3 = lambda f: (0, 0, 0)
    const2 = lambda f: (0, 0)
    return pl.pallas_call(
        _tc_main,
        grid=(F // FB,),
        in_specs=[
            pl.BlockSpec((2, NPAD, NPAD), const3),
            pl.BlockSpec((FB, N, C_IN), lambda f: (f, 0, 0)),
            pl.BlockSpec(W1.shape, const2),
            pl.BlockSpec(b1.shape, const2),
            pl.BlockSpec(W2.shape, const2),
            pl.BlockSpec(b2.shape, const2),
            pl.BlockSpec(Wfc.shape, const2),
            pl.BlockSpec(bfc.shape, const2),
        ],
        out_specs=pl.BlockSpec((1, NPAD), const2),
        out_shape=jax.ShapeDtypeStruct((1, NPAD), jnp.float32),
        scratch_shapes=[
            pltpu.VMEM((NPAD, NPAD), jnp.bfloat16),
            pltpu.VMEM((1, NPAD), jnp.float32),
        ],
        compiler_params=pltpu.CompilerParams(
            dimension_semantics=("arbitrary",)),
    )(a2, xb, W1, b1, W2, b2, Wfc, bfc)


def kernel(x, edge_index, W1, b1, W2, b2, Wfc, bfc):
    ei = jnp.pad(edge_index, ((0, 0), (0, EPAD - E)))
    counts2 = _build_counts(ei[1], ei[0])
    a2 = counts2.reshape(2, NPAD, NPAD)
    xb = x.reshape(F, N, C_IN)
    out = _main(a2, xb, W1.astype(jnp.bfloat16), b1.reshape(1, -1),
                W2.astype(jnp.bfloat16), b2.reshape(1, -1),
                Wfc, bfc.reshape(1, -1))
    return out.reshape(-1)[: Wfc.shape[1]]


# speedup vs baseline: 269.5853x; 1.0770x over previous
import functools

import jax
import jax.numpy as jnp
from jax import lax
from jax.experimental import pallas as pl
from jax.experimental.pallas import tpu as pltpu
from jax.experimental.pallas import tpu_sc as plsc

N = 500
NPAD = 512
F = 50
C_IN = 128
E = 8000
NW = 32
EPW = 256
EPAD = NW * EPW
ACELLS = NPAD * NPAD
SLICE = ACELLS // 16


def _sc_build_counts(dst_hbm, src_hbm, out_hbm, shared, dvec, svec,
                     idx_a, idx_b, val_a, val_b, zbuf):
    c = lax.axis_index("c")
    s = lax.axis_index("s")
    wid = c * 16 + s

    def zbody(i, carry):
        zbuf[pl.ds(i * 16, 16)] = jnp.zeros((16,), jnp.float32)
        return carry
    lax.fori_loop(0, SLICE // 16, zbody, 0)
    pltpu.sync_copy(zbuf, shared.at[pl.ds(s * SLICE, SLICE)])
    plsc.subcore_barrier()

    base = wid * EPW
    pltpu.sync_copy(dst_hbm.at[pl.ds(base, EPW)], dvec)
    pltpu.sync_copy(src_hbm.at[pl.ds(base, EPW)], svec)

    lane = lax.iota(jnp.int32, 16)
    for j in range(EPW // 16):
        d = dvec[pl.ds(j * 16, 16)]
        sv = svec[pl.ds(j * 16, 16)]
        flat = d * NPAD + sv
        gid = base + j * 16 + lane
        val = jnp.where(gid < E, jnp.float32(1.0), jnp.float32(0.0))
        if j < 8:
            idx_a[pl.ds(j * 16, 16)] = flat
            val_a[pl.ds(j * 16, 16)] = val
        else:
            idx_b[pl.ds((j - 8) * 16, 16)] = flat
            val_b[pl.ds((j - 8) * 16, 16)] = val

    pltpu.sync_copy(val_a, shared.at[idx_a], add=True)
    pltpu.sync_copy(val_b, shared.at[idx_b], add=True)
    plsc.subcore_barrier()

    pltpu.sync_copy(shared.at[pl.ds(s * SLICE, SLICE)],
                    out_hbm.at[c, pl.ds(s * SLICE, SLICE)])


@jax.jit
def _build_counts(dst, src):
    mesh = plsc.VectorSubcoreMesh(core_axis_name="c", subcore_axis_name="s")
    return pl.kernel(
        _sc_build_counts,
        out_type=jax.ShapeDtypeStruct((2, ACELLS), jnp.float32),
        mesh=mesh,
        scratch_types=[
            pltpu.MemorySpace.VMEM_SHARED((ACELLS,), jnp.float32),
            pltpu.VMEM((EPW,), jnp.int32),
            pltpu.VMEM((EPW,), jnp.int32),
            pltpu.VMEM((128,), jnp.int32),
            pltpu.VMEM((128,), jnp.int32),
            pltpu.VMEM((128,), jnp.float32),
            pltpu.VMEM((128,), jnp.float32),
            pltpu.VMEM((SLICE,), jnp.float32),
        ],
    )(dst, src)


FB = 5


def _tc_main(a_ref, x_ref, w1_ref, b1_ref, w2_ref, b2_ref, wfc_ref, bfc_ref,
             out_ref, ahat_s, acc_s):
    f = pl.program_id(0)

    @pl.when(f == 0)
    def _init():
        A = a_ref[0] + a_ref[1]
        deg = jnp.sum(A, axis=1) + 1.0
        dinv = lax.rsqrt(deg)
        r = lax.broadcasted_iota(jnp.int32, (NPAD, NPAD), 0)
        cc = lax.broadcasted_iota(jnp.int32, (NPAD, NPAD), 1)
        eye = jnp.where(r == cc, jnp.float32(1.0), jnp.float32(0.0))
        ahat_s[...] = ((A + eye) * dinv[:, None] * dinv[None, :]).astype(
            jnp.bfloat16)
        acc_s[...] = jnp.zeros((1, NPAD), jnp.float32)

    ah = ahat_s[...]
    rr = lax.broadcasted_iota(jnp.int32, (1, NPAD), 1)
    ones_n = jnp.where(rr < N, 1.0, 0.0).astype(jnp.bfloat16)
    xcat = jnp.concatenate([x_ref[i] for i in range(FB)], axis=1)
    xcat = jnp.concatenate(
        [xcat, jnp.zeros((NPAD - N, FB * C_IN), jnp.float32)], axis=0)
    t0 = jnp.dot(ah, xcat.astype(jnp.bfloat16),
                 preferred_element_type=jnp.float32).astype(jnp.bfloat16)
    w1 = w1_ref[...]
    b1 = b1_ref[...]
    h1 = [
        jnp.maximum(
            jnp.dot(t0[:, i * C_IN:(i + 1) * C_IN], w1,
                    preferred_element_type=jnp.float32).astype(jnp.bfloat16)
            + b1, 0)
        for i in range(FB)
    ]
    t1 = jnp.dot(ah, jnp.concatenate(h1, axis=1),
                 preferred_element_type=jnp.float32).astype(jnp.bfloat16)
    w2 = w2_ref[...]
    b2 = b2_ref[...]
    part = jnp.zeros((1, NPAD), jnp.float32)
    for i in range(FB):
        h2 = jnp.maximum(
            jnp.dot(t1[:, i * 256:(i + 1) * 256], w2,
                    preferred_element_type=jnp.float32).astype(jnp.bfloat16)
            + b2, 0)
        part += jnp.dot(ones_n, h2, preferred_element_type=jnp.float32)
    acc_s[...] += part

    @pl.when(f == F // FB - 1)
    def _fin():
        hmean = acc_s[...] / jnp.float32(N * F)
        out_ref[...] = (
            jnp.dot(hmean, wfc_ref[...], preferred_element_type=jnp.float32)
            + bfc_ref[...])


@jax.jit
def _main(a2, xb, W1, b1, W2, b2, Wfc, bfc):
    const3 = lambda f: (0, 0, 0)
    const2 = lambda f: (0, 0)
    return pl.pallas_call(
        _tc_main,
        grid=(F // FB,),
        in_specs=[
            pl.BlockSpec((2, NPAD, NPAD), const3),
            pl.BlockSpec((FB, N, C_IN), lambda f: (f, 0, 0)),
            pl.BlockSpec(W1.shape, const2),
            pl.BlockSpec(b1.shape, const2),
            pl.BlockSpec(W2.shape, const2),
            pl.BlockSpec(b2.shape, const2),
            pl.BlockSpec(Wfc.shape, const2),
            pl.BlockSpec(bfc.shape, const2),
        ],
        out_specs=pl.BlockSpec((1, NPAD), const2),
        out_shape=jax.ShapeDtypeStruct((1, NPAD), jnp.float32),
        scratch_shapes=[
            pltpu.VMEM((NPAD, NPAD), jnp.bfloat16),
            pltpu.VMEM((1, NPAD), jnp.float32),
        ],
        compiler_params=pltpu.CompilerParams(
            dimension_semantics=("arbitrary",)),
    )(a2, xb, W1, b1, W2, b2, Wfc, bfc)


def kernel(x, edge_index, W1, b1, W2, b2, Wfc, bfc):
    ei = jnp.pad(edge_index, ((0, 0), (0, EPAD - E)))
    counts2 = _build_counts(ei[1], ei[0])
    a2 = counts2.reshape(2, NPAD, NPAD)
    xb = x.reshape(F, N, C_IN)
    out = _main(a2, xb, W1.astype(jnp.bfloat16),
                b1.reshape(1, -1).astype(jnp.bfloat16),
                W2.astype(jnp.bfloat16),
                b2.reshape(1, -1).astype(jnp.bfloat16),
                Wfc, bfc.reshape(1, -1))
    return out.reshape(-1)[: Wfc.shape[1]]


# speedup vs baseline: 288.0964x; 1.0687x over previous
import functools

import jax
import jax.numpy as jnp
from jax import lax
from jax.experimental import pallas as pl
from jax.experimental.pallas import tpu as pltpu
from jax.experimental.pallas import tpu_sc as plsc

N = 500
NPAD = 512
F = 50
C_IN = 128
E = 8000
NW = 32
EPW = 256
EPAD = NW * EPW
ACELLS = NPAD * NPAD
SLICE = ACELLS // 16


def _sc_build_counts(dst_hbm, src_hbm, out_hbm, shared, dvec, svec,
                     idx_a, idx_b, val_a, val_b, zbuf):
    c = lax.axis_index("c")
    s = lax.axis_index("s")
    wid = c * 16 + s

    def zbody(i, carry):
        zbuf[pl.ds(i * 16, 16)] = jnp.zeros((16,), jnp.float32)
        return carry
    lax.fori_loop(0, SLICE // 16, zbody, 0)
    pltpu.sync_copy(zbuf, shared.at[pl.ds(s * SLICE, SLICE)])
    plsc.subcore_barrier()

    base = wid * EPW
    pltpu.sync_copy(dst_hbm.at[pl.ds(base, EPW)], dvec)
    pltpu.sync_copy(src_hbm.at[pl.ds(base, EPW)], svec)

    lane = lax.iota(jnp.int32, 16)
    for j in range(EPW // 16):
        d = dvec[pl.ds(j * 16, 16)]
        sv = svec[pl.ds(j * 16, 16)]
        flat = ((sv >> 7) << 16) + (d << 7) + (sv & 127)
        gid = base + j * 16 + lane
        val = jnp.where(gid < E, jnp.float32(1.0), jnp.float32(0.0))
        if j < 8:
            idx_a[pl.ds(j * 16, 16)] = flat
            val_a[pl.ds(j * 16, 16)] = val
        else:
            idx_b[pl.ds((j - 8) * 16, 16)] = flat
            val_b[pl.ds((j - 8) * 16, 16)] = val

    pltpu.sync_copy(val_a, shared.at[idx_a], add=True)
    pltpu.sync_copy(val_b, shared.at[idx_b], add=True)
    plsc.subcore_barrier()

    pltpu.sync_copy(shared.at[pl.ds(s * SLICE, SLICE)],
                    out_hbm.at[c, pl.ds(s * SLICE, SLICE)])


@jax.jit
def _build_counts(dst, src):
    mesh = plsc.VectorSubcoreMesh(core_axis_name="c", subcore_axis_name="s")
    return pl.kernel(
        _sc_build_counts,
        out_type=jax.ShapeDtypeStruct((2, ACELLS), jnp.float32),
        mesh=mesh,
        scratch_types=[
            pltpu.MemorySpace.VMEM_SHARED((ACELLS,), jnp.float32),
            pltpu.VMEM((EPW,), jnp.int32),
            pltpu.VMEM((EPW,), jnp.int32),
            pltpu.VMEM((128,), jnp.int32),
            pltpu.VMEM((128,), jnp.int32),
            pltpu.VMEM((128,), jnp.float32),
            pltpu.VMEM((128,), jnp.float32),
            pltpu.VMEM((SLICE,), jnp.float32),
        ],
    )(dst, src)


FB = 10


def _tc_main(a_ref, x_ref, w1_ref, b1_ref, w2_ref, b2_ref, wfc_ref, bfc_ref,
             out_ref, ahat_s, acc_s):
    f = pl.program_id(0)

    @pl.when(f == 0)
    def _init():
        A = jnp.concatenate(
            [a_ref[0, c] + a_ref[1, c] for c in range(4)], axis=1)
        deg = jnp.sum(A, axis=1) + 1.0
        dinv = lax.rsqrt(deg)
        r = lax.broadcasted_iota(jnp.int32, (NPAD, NPAD), 0)
        cc = lax.broadcasted_iota(jnp.int32, (NPAD, NPAD), 1)
        eye = jnp.where(r == cc, jnp.float32(1.0), jnp.float32(0.0))
        ahat_s[...] = ((A + eye) * dinv[:, None] * dinv[None, :]).astype(
            jnp.bfloat16)
        acc_s[...] = jnp.zeros((1, NPAD), jnp.float32)

    ah = ahat_s[...]
    rr = lax.broadcasted_iota(jnp.int32, (1, NPAD), 1)
    ones_n = jnp.where(rr < N, 1.0, 0.0).astype(jnp.bfloat16)
    xflat = x_ref[...].reshape(10 * FB * 50, C_IN)
    xcat = jnp.concatenate(
        [xflat[k * N:(k + 1) * N] for k in range(FB)], axis=1)
    xcat = jnp.concatenate(
        [xcat, jnp.zeros((NPAD - N, FB * C_IN), jnp.float32)], axis=0)
    t0 = jnp.dot(ah, xcat.astype(jnp.bfloat16),
                 preferred_element_type=jnp.float32).astype(jnp.bfloat16)
    w1 = w1_ref[...]
    b1 = b1_ref[...]
    h1 = [
        jnp.maximum(
            jnp.dot(t0[:, i * C_IN:(i + 1) * C_IN], w1,
                    preferred_element_type=jnp.float32).astype(jnp.bfloat16)
            + b1, 0)
        for i in range(FB)
    ]
    t1 = jnp.dot(ah, jnp.concatenate(h1, axis=1),
                 preferred_element_type=jnp.float32).astype(jnp.bfloat16)
    w2 = w2_ref[...]
    b2 = b2_ref[...]
    part = jnp.zeros((1, NPAD), jnp.float32)
    for i in range(FB):
        h2 = jnp.maximum(
            jnp.dot(t1[:, i * 256:(i + 1) * 256], w2,
                    preferred_element_type=jnp.float32).astype(jnp.bfloat16)
            + b2, 0)
        part += jnp.dot(ones_n, h2, preferred_element_type=jnp.float32)
    acc_s[...] += part

    @pl.when(f == F // FB - 1)
    def _fin():
        hmean = acc_s[...] / jnp.float32(N * F)
        out_ref[...] = (
            jnp.dot(hmean, wfc_ref[...], preferred_element_type=jnp.float32)
            + bfc_ref[...])


@jax.jit
def _main(a2, xb, W1, b1, W2, b2, Wfc, bfc):
    const4 = lambda f: (0, 0, 0, 0)
    const2 = lambda f: (0, 0)
    return pl.pallas_call(
        _tc_main,
        grid=(F // FB,),
        in_specs=[
            pl.BlockSpec((2, 4, NPAD, C_IN), const4),
            pl.BlockSpec((10 * FB, F, C_IN), lambda f: (f, 0, 0)),
            pl.BlockSpec(W1.shape, const2),
            pl.BlockSpec(b1.shape, const2),
            pl.BlockSpec(W2.shape, const2),
            pl.BlockSpec(b2.shape, const2),
            pl.BlockSpec(Wfc.shape, const2),
            pl.BlockSpec(bfc.shape, const2),
        ],
        out_specs=pl.BlockSpec((1, NPAD), const2),
        out_shape=jax.ShapeDtypeStruct((1, NPAD), jnp.float32),
        scratch_shapes=[
            pltpu.VMEM((NPAD, NPAD), jnp.bfloat16),
            pltpu.VMEM((1, NPAD), jnp.float32),
        ],
        compiler_params=pltpu.CompilerParams(
            dimension_semantics=("arbitrary",)),
    )(a2, xb, W1, b1, W2, b2, Wfc, bfc)


def kernel(x, edge_index, W1, b1, W2, b2, Wfc, bfc):
    ei = jnp.pad(edge_index, ((0, 0), (0, EPAD - E)))
    counts2 = _build_counts(ei[1], ei[0])
    a2 = counts2.reshape(2, 4, NPAD, C_IN)
    out = _main(a2, x, W1.astype(jnp.bfloat16),
                b1.reshape(1, -1).astype(jnp.bfloat16),
                W2.astype(jnp.bfloat16),
                b2.reshape(1, -1).astype(jnp.bfloat16),
                Wfc, bfc.reshape(1, -1))
    return out.reshape(-1)[: Wfc.shape[1]]


# speedup vs baseline: 322.6617x; 1.1200x over previous
import functools

import jax
import jax.numpy as jnp
from jax import lax
from jax.experimental import pallas as pl
from jax.experimental.pallas import tpu as pltpu
from jax.experimental.pallas import tpu_sc as plsc

N = 500
NPAD = 512
F = 50
C_IN = 128
E = 8000
NW = 32
EPW = 256
EPAD = NW * EPW
ACELLS = NPAD * NPAD
SLICE = ACELLS // 16


EPR = E // NW


def _sc_build_counts(dst_hbm, src_hbm, out_hbm, shared, dvec, svec,
                     idx_a, idx_b, val_a, val_b, zbuf):
    c = lax.axis_index("c")
    s = lax.axis_index("s")
    wid = c * 16 + s

    def zbody(i, carry):
        zbuf[pl.ds(i * 16, 16)] = jnp.zeros((16,), jnp.float32)
        return carry
    lax.fori_loop(0, SLICE // 16, zbody, 0)
    pltpu.sync_copy(zbuf, shared.at[pl.ds(s * SLICE, SLICE)])
    plsc.subcore_barrier()

    lo = wid * EPR
    base = pl.multiple_of(lo - (lo & 7), 8)
    pltpu.sync_copy(dst_hbm.at[pl.ds(base, EPW)], dvec)
    pltpu.sync_copy(src_hbm.at[pl.ds(base, EPW)], svec)

    lane = lax.iota(jnp.int32, 16)
    for j in range(EPW // 16):
        d = dvec[pl.ds(j * 16, 16)]
        sv = svec[pl.ds(j * 16, 16)]
        flat = ((sv >> 7) << 16) + (d << 7) + (sv & 127)
        gid = base + j * 16 + lane
        val = jnp.where((gid >= lo) & (gid < lo + EPR),
                        jnp.float32(1.0), jnp.float32(0.0))
        if j < 8:
            idx_a[pl.ds(j * 16, 16)] = flat
            val_a[pl.ds(j * 16, 16)] = val
        else:
            idx_b[pl.ds((j - 8) * 16, 16)] = flat
            val_b[pl.ds((j - 8) * 16, 16)] = val

    pltpu.sync_copy(val_a, shared.at[idx_a], add=True)
    pltpu.sync_copy(val_b, shared.at[idx_b], add=True)
    plsc.subcore_barrier()

    pltpu.sync_copy(shared.at[pl.ds(s * SLICE, SLICE)],
                    out_hbm.at[pl.ds(wid * SLICE, SLICE)])


@jax.jit
def _build_counts(dst, src):
    mesh = plsc.VectorSubcoreMesh(core_axis_name="c", subcore_axis_name="s")
    return pl.kernel(
        _sc_build_counts,
        out_type=jax.ShapeDtypeStruct((2 * ACELLS,), jnp.float32),
        mesh=mesh,
        scratch_types=[
            pltpu.MemorySpace.VMEM_SHARED((ACELLS,), jnp.float32),
            pltpu.VMEM((EPW,), jnp.int32),
            pltpu.VMEM((EPW,), jnp.int32),
            pltpu.VMEM((128,), jnp.int32),
            pltpu.VMEM((128,), jnp.int32),
            pltpu.VMEM((128,), jnp.float32),
            pltpu.VMEM((128,), jnp.float32),
            pltpu.VMEM((SLICE,), jnp.float32),
        ],
    )(dst, src)


FB = 10


def _tc_main(a_ref, x_ref, w1_ref, b1_ref, w2_ref, b2_ref, wfc_ref, bfc_ref,
             out_ref, ahat_s, acc_s, w1_s, w2_s):
    f = pl.program_id(0)

    @pl.when(f == 0)
    def _init():
        A = jnp.concatenate(
            [a_ref[0, c] + a_ref[1, c] for c in range(4)], axis=1)
        deg = jnp.sum(A, axis=1) + 1.0
        dinv = lax.rsqrt(deg)
        r = lax.broadcasted_iota(jnp.int32, (NPAD, NPAD), 0)
        cc = lax.broadcasted_iota(jnp.int32, (NPAD, NPAD), 1)
        eye = jnp.where(r == cc, jnp.float32(1.0), jnp.float32(0.0))
        ahat_s[...] = ((A + eye) * dinv[:, None] * dinv[None, :]).astype(
            jnp.bfloat16)
        acc_s[...] = jnp.zeros((1, NPAD), jnp.float32)
        w1_s[...] = w1_ref[...].astype(jnp.bfloat16)
        w2_s[...] = w2_ref[...].astype(jnp.bfloat16)

    ah = ahat_s[...]
    rr = lax.broadcasted_iota(jnp.int32, (1, NPAD), 1)
    ones_n = jnp.where(rr < N, 1.0, 0.0).astype(jnp.bfloat16)
    xflat = x_ref[...].reshape(10 * FB * 50, C_IN)
    xcat = jnp.concatenate(
        [xflat[k * N:(k + 1) * N] for k in range(FB)], axis=1)
    xcat = jnp.concatenate(
        [xcat, jnp.zeros((NPAD - N, FB * C_IN), jnp.float32)], axis=0)
    t0 = jnp.dot(ah, xcat.astype(jnp.bfloat16),
                 preferred_element_type=jnp.float32).astype(jnp.bfloat16)
    w1 = w1_s[...]
    b1 = b1_ref[...].astype(jnp.bfloat16)
    h1 = [
        jnp.maximum(
            jnp.dot(t0[:, i * C_IN:(i + 1) * C_IN], w1,
                    preferred_element_type=jnp.float32).astype(jnp.bfloat16)
            + b1, 0)
        for i in range(FB)
    ]
    t1 = jnp.dot(ah, jnp.concatenate(h1, axis=1),
                 preferred_element_type=jnp.float32).astype(jnp.bfloat16)
    w2 = w2_s[...]
    b2 = b2_ref[...].astype(jnp.bfloat16)
    part = jnp.zeros((1, NPAD), jnp.float32)
    for i in range(FB):
        h2 = jnp.maximum(
            jnp.dot(t1[:, i * 256:(i + 1) * 256], w2,
                    preferred_element_type=jnp.float32).astype(jnp.bfloat16)
            + b2, 0)
        part += jnp.dot(ones_n, h2, preferred_element_type=jnp.float32)
    acc_s[...] += part

    @pl.when(f == F // FB - 1)
    def _fin():
        hmean = acc_s[...] / jnp.float32(N * F)
        out_ref[...] = (
            jnp.dot(hmean, wfc_ref[...], preferred_element_type=jnp.float32)
            + bfc_ref[...])


@jax.jit
def _main(a2, xb, W1, b1, W2, b2, Wfc, bfc):
    const4 = lambda f: (0, 0, 0, 0)
    const2 = lambda f: (0, 0)
    return pl.pallas_call(
        _tc_main,
        grid=(F // FB,),
        in_specs=[
            pl.BlockSpec((2, 4, NPAD, C_IN), const4),
            pl.BlockSpec((10 * FB, F, C_IN), lambda f: (f, 0, 0)),
            pl.BlockSpec(W1.shape, const2),
            pl.BlockSpec(b1.shape, const2),
            pl.BlockSpec(W2.shape, const2),
            pl.BlockSpec(b2.shape, const2),
            pl.BlockSpec(Wfc.shape, const2),
            pl.BlockSpec(bfc.shape, const2),
        ],
        out_specs=pl.BlockSpec((1, NPAD), const2),
        out_shape=jax.ShapeDtypeStruct((1, NPAD), jnp.float32),
        scratch_shapes=[
            pltpu.VMEM((NPAD, NPAD), jnp.bfloat16),
            pltpu.VMEM((1, NPAD), jnp.float32),
            pltpu.VMEM((C_IN, 256), jnp.bfloat16),
            pltpu.VMEM((256, 512), jnp.bfloat16),
        ],
        compiler_params=pltpu.CompilerParams(
            dimension_semantics=("arbitrary",)),
    )(a2, xb, W1, b1, W2, b2, Wfc, bfc)


def kernel(x, edge_index, W1, b1, W2, b2, Wfc, bfc):
    counts2 = _build_counts(edge_index[1], edge_index[0])
    a2 = counts2.reshape(2, 4, NPAD, C_IN)
    out = _main(a2, x, W1, b1.reshape(1, -1), W2, b2.reshape(1, -1),
                Wfc, bfc.reshape(1, -1))
    return out.reshape(-1)[: Wfc.shape[1]]


# speedup vs baseline: 342.2525x; 1.0607x over previous
import functools

import jax
import jax.numpy as jnp
from jax import lax
from jax.experimental import pallas as pl
from jax.experimental.pallas import tpu as pltpu
from jax.experimental.pallas import tpu_sc as plsc

N = 500
NPAD = 512
F = 50
C_IN = 128
E = 8000
NW = 32
EPW = 256
EPAD = NW * EPW
ACELLS = NPAD * NPAD
SLICE = ACELLS // 16


EPR = E // NW


def _sc_build_counts(dst_hbm, src_hbm, out_hbm, shared, dvec, svec,
                     idx_a, idx_b, val_a, val_b, zbuf):
    c = lax.axis_index("c")
    s = lax.axis_index("s")
    wid = c * 16 + s

    def zbody(i, carry):
        zbuf[pl.ds(i * 16, 16)] = jnp.zeros((16,), jnp.float32)
        return carry
    lax.fori_loop(0, SLICE // 16, zbody, 0)
    pltpu.sync_copy(zbuf, shared.at[pl.ds(s * SLICE, SLICE)])
    plsc.subcore_barrier()

    lo = wid * EPR
    base = pl.multiple_of(lo - (lo & 7), 8)
    pltpu.sync_copy(dst_hbm.at[pl.ds(base, EPW)], dvec)
    pltpu.sync_copy(src_hbm.at[pl.ds(base, EPW)], svec)

    lane = lax.iota(jnp.int32, 16)
    for j in range(EPW // 16):
        d = dvec[pl.ds(j * 16, 16)]
        sv = svec[pl.ds(j * 16, 16)]
        flat = ((sv >> 7) << 16) + (d << 7) + (sv & 127)
        gid = base + j * 16 + lane
        val = jnp.where((gid >= lo) & (gid < lo + EPR),
                        jnp.float32(1.0), jnp.float32(0.0))
        if j < 8:
            idx_a[pl.ds(j * 16, 16)] = flat
            val_a[pl.ds(j * 16, 16)] = val
        else:
            idx_b[pl.ds((j - 8) * 16, 16)] = flat
            val_b[pl.ds((j - 8) * 16, 16)] = val

    pltpu.sync_copy(val_a, shared.at[idx_a], add=True)
    pltpu.sync_copy(val_b, shared.at[idx_b], add=True)
    plsc.subcore_barrier()

    pltpu.sync_copy(shared.at[pl.ds(s * SLICE, SLICE)],
                    out_hbm.at[pl.ds(wid * SLICE, SLICE)])


@jax.jit
def _build_counts(dst, src):
    mesh = plsc.VectorSubcoreMesh(core_axis_name="c", subcore_axis_name="s")
    return pl.kernel(
        _sc_build_counts,
        out_type=jax.ShapeDtypeStruct((2 * ACELLS,), jnp.float32),
        mesh=mesh,
        scratch_types=[
            pltpu.MemorySpace.VMEM_SHARED((ACELLS,), jnp.float32),
            pltpu.VMEM((EPW,), jnp.int32),
            pltpu.VMEM((EPW,), jnp.int32),
            pltpu.VMEM((128,), jnp.int32),
            pltpu.VMEM((128,), jnp.int32),
            pltpu.VMEM((128,), jnp.float32),
            pltpu.VMEM((128,), jnp.float32),
            pltpu.VMEM((SLICE,), jnp.float32),
        ],
    )(dst, src)


FB = 25


def _tc_main(a_ref, x_ref, w1_ref, b1_ref, w2_ref, b2_ref, wfc_ref, bfc_ref,
             out_ref, ahat_s, acc_s, w1_s, w2_s):
    f = pl.program_id(0)

    @pl.when(f == 0)
    def _init():
        A = jnp.concatenate(
            [a_ref[0, c] + a_ref[1, c] for c in range(4)], axis=1)
        deg = jnp.sum(A, axis=1) + 1.0
        dinv = lax.rsqrt(deg)
        r = lax.broadcasted_iota(jnp.int32, (NPAD, NPAD), 0)
        cc = lax.broadcasted_iota(jnp.int32, (NPAD, NPAD), 1)
        eye = jnp.where(r == cc, jnp.float32(1.0), jnp.float32(0.0))
        ahat_s[...] = ((A + eye) * dinv[:, None] * dinv[None, :]).astype(
            jnp.bfloat16)
        acc_s[...] = jnp.zeros((1, NPAD), jnp.float32)
        w1_s[...] = w1_ref[...].astype(jnp.bfloat16)
        w2_s[...] = w2_ref[...].astype(jnp.bfloat16)

    ah = ahat_s[...]
    rr = lax.broadcasted_iota(jnp.int32, (1, NPAD), 1)
    ones_n = jnp.where(rr < N, 1.0, 0.0).astype(jnp.bfloat16)
    xflat = x_ref[...].reshape(10 * FB * 50, C_IN)
    xcat = jnp.concatenate(
        [xflat[k * N:(k + 1) * N] for k in range(FB)], axis=1)
    xcat = jnp.concatenate(
        [xcat, jnp.zeros((NPAD - N, FB * C_IN), jnp.float32)], axis=0)
    t0 = jnp.dot(ah, xcat.astype(jnp.bfloat16),
                 preferred_element_type=jnp.float32).astype(jnp.bfloat16)
    w1 = w1_s[...]
    b1 = b1_ref[...].astype(jnp.bfloat16)
    h1 = [
        jnp.maximum(
            jnp.dot(t0[:, i * C_IN:(i + 1) * C_IN], w1,
                    preferred_element_type=jnp.float32).astype(jnp.bfloat16)
            + b1, 0)
        for i in range(FB)
    ]
    t1 = jnp.dot(ah, jnp.concatenate(h1, axis=1),
                 preferred_element_type=jnp.float32).astype(jnp.bfloat16)
    w2 = w2_s[...]
    b2 = b2_ref[...].astype(jnp.bfloat16)
    part = jnp.zeros((1, NPAD), jnp.float32)
    for i in range(FB):
        h2 = jnp.maximum(
            jnp.dot(t1[:, i * 256:(i + 1) * 256], w2,
                    preferred_element_type=jnp.float32).astype(jnp.bfloat16)
            + b2, 0)
        part += jnp.dot(ones_n, h2, preferred_element_type=jnp.float32)
    acc_s[...] += part

    @pl.when(f == F // FB - 1)
    def _fin():
        hmean = acc_s[...] / jnp.float32(N * F)
        out_ref[...] = (
            jnp.dot(hmean, wfc_ref[...], preferred_element_type=jnp.float32)
            + bfc_ref[...])


@jax.jit
def _main(a2, xb, W1, b1, W2, b2, Wfc, bfc):
    const4 = lambda f: (0, 0, 0, 0)
    const2 = lambda f: (0, 0)
    return pl.pallas_call(
        _tc_main,
        grid=(F // FB,),
        in_specs=[
            pl.BlockSpec((2, 4, NPAD, C_IN), const4),
            pl.BlockSpec((10 * FB, F, C_IN), lambda f: (f, 0, 0)),
            pl.BlockSpec(W1.shape, const2),
            pl.BlockSpec(b1.shape, const2),
            pl.BlockSpec(W2.shape, const2),
            pl.BlockSpec(b2.shape, const2),
            pl.BlockSpec(Wfc.shape, const2),
            pl.BlockSpec(bfc.shape, const2),
        ],
        out_specs=pl.BlockSpec((1, NPAD), const2),
        out_shape=jax.ShapeDtypeStruct((1, NPAD), jnp.float32),
        scratch_shapes=[
            pltpu.VMEM((NPAD, NPAD), jnp.bfloat16),
            pltpu.VMEM((1, NPAD), jnp.float32),
            pltpu.VMEM((C_IN, 256), jnp.bfloat16),
            pltpu.VMEM((256, 512), jnp.bfloat16),
        ],
        compiler_params=pltpu.CompilerParams(
            dimension_semantics=("arbitrary",)),
    )(a2, xb, W1, b1, W2, b2, Wfc, bfc)


def kernel(x, edge_index, W1, b1, W2, b2, Wfc, bfc):
    counts2 = _build_counts(edge_index[1], edge_index[0])
    a2 = counts2.reshape(2, 4, NPAD, C_IN)
    out = _main(a2, x, W1, b1.reshape(1, -1), W2, b2.reshape(1, -1),
                Wfc, bfc.reshape(1, -1))
    return out.reshape(-1)[: Wfc.shape[1]]


# speedup vs baseline: 346.7743x; 1.0132x over previous
import functools

import jax
import jax.numpy as jnp
from jax import lax
from jax.experimental import pallas as pl
from jax.experimental.pallas import tpu as pltpu
from jax.experimental.pallas import tpu_sc as plsc

N = 500
NPAD = 512
F = 50
C_IN = 128
E = 8000
NW = 32
EPW = 256
EPAD = NW * EPW
ACELLS = NPAD * NPAD
SLICE = ACELLS // 16


EPR = E // NW


def _sc_build_counts(dst_hbm, src_hbm, out_hbm, shared, dvec, svec,
                     idx_a, idx_b, val_a, val_b, zbuf):
    c = lax.axis_index("c")
    s = lax.axis_index("s")
    wid = c * 16 + s

    def zbody(i, carry):
        zbuf[pl.ds(i * 16, 16)] = jnp.zeros((16,), jnp.float32)
        return carry
    lax.fori_loop(0, SLICE // 16, zbody, 0)
    pltpu.sync_copy(zbuf, shared.at[pl.ds(s * SLICE, SLICE)])
    plsc.subcore_barrier()

    lo = wid * EPR
    base = pl.multiple_of(lo - (lo & 7), 8)
    pltpu.sync_copy(dst_hbm.at[pl.ds(base, EPW)], dvec)
    pltpu.sync_copy(src_hbm.at[pl.ds(base, EPW)], svec)

    lane = lax.iota(jnp.int32, 16)
    for j in range(EPW // 16):
        d = dvec[pl.ds(j * 16, 16)]
        sv = svec[pl.ds(j * 16, 16)]
        flat = ((sv >> 7) << 16) + (d << 7) + (sv & 127)
        gid = base + j * 16 + lane
        val = jnp.where((gid >= lo) & (gid < lo + EPR),
                        jnp.float32(1.0), jnp.float32(0.0))
        if j < 8:
            idx_a[pl.ds(j * 16, 16)] = flat
            val_a[pl.ds(j * 16, 16)] = val
        else:
            idx_b[pl.ds((j - 8) * 16, 16)] = flat
            val_b[pl.ds((j - 8) * 16, 16)] = val

    pltpu.sync_copy(val_a, shared.at[idx_a], add=True)
    pltpu.sync_copy(val_b, shared.at[idx_b], add=True)
    plsc.subcore_barrier()

    pltpu.sync_copy(shared.at[pl.ds(s * SLICE, SLICE)],
                    out_hbm.at[pl.ds(wid * SLICE, SLICE)])


@jax.jit
def _build_counts(dst, src):
    mesh = plsc.VectorSubcoreMesh(core_axis_name="c", subcore_axis_name="s")
    return pl.kernel(
        _sc_build_counts,
        out_type=jax.ShapeDtypeStruct((2 * ACELLS,), jnp.float32),
        mesh=mesh,
        scratch_types=[
            pltpu.MemorySpace.VMEM_SHARED((ACELLS,), jnp.float32),
            pltpu.VMEM((EPW,), jnp.int32),
            pltpu.VMEM((EPW,), jnp.int32),
            pltpu.VMEM((128,), jnp.int32),
            pltpu.VMEM((128,), jnp.int32),
            pltpu.VMEM((128,), jnp.float32),
            pltpu.VMEM((128,), jnp.float32),
            pltpu.VMEM((SLICE,), jnp.float32),
        ],
    )(dst, src)


FB = 25
GB = 5


def _tc_main(a_ref, x_ref, w1_ref, b1_ref, w2_ref, b2_ref, wfc_ref, bfc_ref,
             out_ref, ahat_s, acc_s, w1_s, w2_s):
    f = pl.program_id(0)

    @pl.when(f == 0)
    def _init():
        A = jnp.concatenate(
            [a_ref[0, c] + a_ref[1, c] for c in range(4)], axis=1)
        deg = jnp.sum(A, axis=1) + 1.0
        dinv = lax.rsqrt(deg)
        r = lax.broadcasted_iota(jnp.int32, (NPAD, NPAD), 0)
        cc = lax.broadcasted_iota(jnp.int32, (NPAD, NPAD), 1)
        eye = jnp.where(r == cc, jnp.float32(1.0), jnp.float32(0.0))
        ahat_s[...] = ((A + eye) * dinv[:, None] * dinv[None, :]).astype(
            jnp.bfloat16)
        acc_s[...] = jnp.zeros((1, NPAD), jnp.float32)
        w1_s[...] = w1_ref[...].astype(jnp.bfloat16)
        w2_s[...] = w2_ref[...].astype(jnp.bfloat16)

    ah = ahat_s[...]
    rr = lax.broadcasted_iota(jnp.int32, (1, NPAD), 1)
    ones_n = jnp.where(rr < N, 1.0, 0.0).astype(jnp.bfloat16)
    xflat = x_ref[...].reshape(10 * FB * 50, C_IN)
    w1 = w1_s[...]
    b1 = b1_ref[...].astype(jnp.bfloat16)
    w2 = w2_s[...]
    b2 = b2_ref[...].astype(jnp.bfloat16)
    zpad = jnp.zeros((NPAD - N, GB * C_IN), jnp.float32)
    part = jnp.zeros((1, NPAD), jnp.float32)
    for g in range(FB // GB):
        xcat = jnp.concatenate(
            [xflat[(g * GB + k) * N:(g * GB + k + 1) * N] for k in range(GB)],
            axis=1)
        xcat = jnp.concatenate([xcat, zpad], axis=0)
        t0 = jnp.dot(ah, xcat.astype(jnp.bfloat16),
                     preferred_element_type=jnp.float32).astype(jnp.bfloat16)
        h1 = [
            jnp.maximum(
                jnp.dot(t0[:, k * C_IN:(k + 1) * C_IN], w1,
                        preferred_element_type=jnp.float32).astype(
                            jnp.bfloat16) + b1, 0)
            for k in range(GB)
        ]
        t1 = jnp.dot(ah, jnp.concatenate(h1, axis=1),
                     preferred_element_type=jnp.float32).astype(jnp.bfloat16)
        for k in range(GB):
            h2 = jnp.maximum(
                jnp.dot(t1[:, k * 256:(k + 1) * 256], w2,
                        preferred_element_type=jnp.float32).astype(
                            jnp.bfloat16) + b2, 0)
            part += jnp.dot(ones_n, h2, preferred_element_type=jnp.float32)
    acc_s[...] += part

    @pl.when(f == F // FB - 1)
    def _fin():
        hmean = acc_s[...] / jnp.float32(N * F)
        out_ref[...] = (
            jnp.dot(hmean, wfc_ref[...], preferred_element_type=jnp.float32)
            + bfc_ref[...])


@jax.jit
def _main(a2, xb, W1, b1, W2, b2, Wfc, bfc):
    const4 = lambda f: (0, 0, 0, 0)
    const2 = lambda f: (0, 0)
    return pl.pallas_call(
        _tc_main,
        grid=(F // FB,),
        in_specs=[
            pl.BlockSpec((2, 4, NPAD, C_IN), const4),
            pl.BlockSpec((10 * FB, F, C_IN), lambda f: (f, 0, 0)),
            pl.BlockSpec(W1.shape, const2),
            pl.BlockSpec(b1.shape, const2),
            pl.BlockSpec(W2.shape, const2),
            pl.BlockSpec(b2.shape, const2),
            pl.BlockSpec(Wfc.shape, const2),
            pl.BlockSpec(bfc.shape, const2),
        ],
        out_specs=pl.BlockSpec((1, NPAD), const2),
        out_shape=jax.ShapeDtypeStruct((1, NPAD), jnp.float32),
        scratch_shapes=[
            pltpu.VMEM((NPAD, NPAD), jnp.bfloat16),
            pltpu.VMEM((1, NPAD), jnp.float32),
            pltpu.VMEM((C_IN, 256), jnp.bfloat16),
            pltpu.VMEM((256, 512), jnp.bfloat16),
        ],
        compiler_params=pltpu.CompilerParams(
            dimension_semantics=("arbitrary",)),
    )(a2, xb, W1, b1, W2, b2, Wfc, bfc)


def kernel(x, edge_index, W1, b1, W2, b2, Wfc, bfc):
    counts2 = _build_counts(edge_index[1], edge_index[0])
    a2 = counts2.reshape(2, 4, NPAD, C_IN)
    out = _main(a2, x, W1, b1.reshape(1, -1), W2, b2.reshape(1, -1),
                Wfc, bfc.reshape(1, -1))
    return out.reshape(-1)[: Wfc.shape[1]]
